# baseline (speedup 1.0000x reference)
import jax
import jax.numpy as jnp
from jax.experimental import pallas as pl
from jax.experimental.pallas import tpu as pltpu

CH = 2048
DCH = 1024


def _top2(logits, g_ref, i_ref):
    e = logits.shape[-1]
    iota = jax.lax.broadcasted_iota(jnp.int32, logits.shape, 1)
    m1 = jnp.max(logits, axis=-1, keepdims=True)
    i1 = jnp.min(jnp.where(logits == m1, iota, e), axis=-1, keepdims=True)
    masked = jnp.where(iota == i1, -jnp.inf, logits)
    m2 = jnp.max(masked, axis=-1, keepdims=True)
    i2 = jnp.min(jnp.where(masked == m2, iota, e), axis=-1, keepdims=True)
    t = jnp.exp(m2 - m1)
    g1 = 1.0 / (1.0 + t)
    g2 = t * g1
    g_ref[...] = jnp.concatenate([g1, g2], axis=-1)
    i_ref[...] = jnp.concatenate([i1, i2], axis=-1)


def _router_body(x_ref, w_ref, g_ref, i_ref, acc_ref):
    dk = pl.program_id(1)
    nd = pl.num_programs(1)
    partial = jax.lax.dot_general(
        x_ref[...], w_ref[...], (((1,), (1,)), ((), ())),
        preferred_element_type=jnp.float32,
    )

    @pl.when(dk == 0)
    def _():
        acc_ref[...] = partial

    @pl.when(dk != 0)
    def _():
        acc_ref[...] += partial

    @pl.when(dk == nd - 1)
    def _():
        _top2(acc_ref[...], g_ref, i_ref)


@jax.jit
def _router(x, W):
    n, d = x.shape
    num_e = W.shape[0]
    grid = (n // CH, d // DCH)
    gates, idx = pl.pallas_call(
        _router_body,
        grid=grid,
        in_specs=[
            pl.BlockSpec((CH, DCH), lambda t, dk: (t, dk)),
            pl.BlockSpec((num_e, DCH), lambda t, dk: (0, dk)),
        ],
        out_specs=[
            pl.BlockSpec((CH, 2), lambda t, dk: (t, 0)),
            pl.BlockSpec((CH, 2), lambda t, dk: (t, 0)),
        ],
        out_shape=[
            jax.ShapeDtypeStruct((n, 2), jnp.float32),
            jax.ShapeDtypeStruct((n, 2), jnp.int32),
        ],
        scratch_shapes=[
            pltpu.VMEM((CH, num_e), jnp.float32),
        ],
        compiler_params=pltpu.CompilerParams(
            dimension_semantics=("arbitrary", "arbitrary"),
        ),
    )(x, W)
    return gates, idx


def kernel(x, W):
    gates, idx = _router(x, W)
    return gates, idx, jnp.zeros((), dtype=jnp.float32)


# speedup vs baseline: 1.0097x; 1.0097x over previous
import jax
import jax.numpy as jnp
from jax.experimental import pallas as pl
from jax.experimental.pallas import tpu as pltpu

CH = 2048
NBUF = 2
QS = 4
QCH = CH // QS


def _top2(logits, g_ref, i_ref, row0):
    e = logits.shape[-1]
    iota = jax.lax.broadcasted_iota(jnp.int32, logits.shape, 1)
    m1 = jnp.max(logits, axis=-1, keepdims=True)
    i1 = jnp.min(jnp.where(logits == m1, iota, e), axis=-1, keepdims=True)
    masked = jnp.where(iota == i1, -jnp.inf, logits)
    m2 = jnp.max(masked, axis=-1, keepdims=True)
    i2 = jnp.min(jnp.where(masked == m2, iota, e), axis=-1, keepdims=True)
    t = jnp.exp(m2 - m1)
    g1 = 1.0 / (1.0 + t)
    g2 = t * g1
    g_ref[row0:row0 + QCH, :] = jnp.concatenate([g1, g2], axis=-1)
    i_ref[row0:row0 + QCH, :] = jnp.concatenate([i1, i2], axis=-1)


def _router_body(x_hbm, w_ref, g_ref, i_ref, xbuf, sems):
    step = pl.program_id(0)
    nsteps = pl.num_programs(0)

    def copy_obj(chunk, buf, q):
        return pltpu.make_async_copy(
            x_hbm.at[pl.ds(chunk * CH + q * QCH, QCH), :],
            xbuf.at[buf, pl.ds(q * QCH, QCH), :],
            sems.at[buf, q],
        )

    @pl.when(step == 0)
    def _():
        for j in range(NBUF):
            for q in range(QS):
                copy_obj(j, j, q).start()

    buf = jax.lax.rem(step, NBUF)
    w = w_ref[...]
    for j in range(NBUF):
        @pl.when(buf == j)
        def _(j=j):
            for q in range(QS):
                copy_obj(step, j, q).wait()
                logits = jax.lax.dot_general(
                    xbuf[j, q * QCH:(q + 1) * QCH, :], w,
                    (((1,), (1,)), ((), ())),
                    preferred_element_type=jnp.float32,
                )
                _top2(logits, g_ref, i_ref, q * QCH)

                @pl.when(step + NBUF < nsteps)
                def _():
                    copy_obj(step + NBUF, j, q).start()


@jax.jit
def _router(x, W):
    n, d = x.shape
    num_e = W.shape[0]
    grid = (n // CH,)
    gates, idx = pl.pallas_call(
        _router_body,
        grid=grid,
        in_specs=[
            pl.BlockSpec(memory_space=pl.ANY),
            pl.BlockSpec((num_e, d), lambda t: (0, 0)),
        ],
        out_specs=[
            pl.BlockSpec((CH, 2), lambda t: (t, 0)),
            pl.BlockSpec((CH, 2), lambda t: (t, 0)),
        ],
        out_shape=[
            jax.ShapeDtypeStruct((n, 2), jnp.float32),
            jax.ShapeDtypeStruct((n, 2), jnp.int32),
        ],
        scratch_shapes=[
            pltpu.VMEM((NBUF, CH, d), jnp.float32),
            pltpu.SemaphoreType.DMA((NBUF, QS)),
        ],
        compiler_params=pltpu.CompilerParams(
            dimension_semantics=("arbitrary",),
        ),
    )(x, W)
    return gates, idx


def kernel(x, W):
    gates, idx = _router(x, W)
    return gates, idx, jnp.zeros((), dtype=jnp.float32)


# speedup vs baseline: 1.1947x; 1.1832x over previous
import jax
import jax.numpy as jnp
from jax.experimental import pallas as pl
from jax.experimental.pallas import tpu as pltpu

CH = 2048
NBUF = 3
QS = 1
QCH = CH // QS


def _top2(logits, g_ref, i_ref, row0):
    e = logits.shape[-1]
    iota = jax.lax.broadcasted_iota(jnp.int32, logits.shape, 1)
    m1 = jnp.max(logits, axis=-1, keepdims=True)
    i1 = jnp.min(jnp.where(logits == m1, iota, e), axis=-1, keepdims=True)
    masked = jnp.where(iota == i1, -jnp.inf, logits)
    m2 = jnp.max(masked, axis=-1, keepdims=True)
    i2 = jnp.min(jnp.where(masked == m2, iota, e), axis=-1, keepdims=True)
    t = jnp.exp(m2 - m1)
    g1 = 1.0 / (1.0 + t)
    g2 = t * g1
    g_ref[row0:row0 + QCH, :] = jnp.concatenate([g1, g2], axis=-1)
    i_ref[row0:row0 + QCH, :] = jnp.concatenate([i1, i2], axis=-1)


def _router_body(x_hbm, w_ref, g_ref, i_ref, xbuf, sems):
    step = pl.program_id(0)
    nsteps = pl.num_programs(0)

    def copy_obj(chunk, buf, q):
        return pltpu.make_async_copy(
            x_hbm.at[pl.ds(chunk * CH + q * QCH, QCH), :],
            xbuf.at[buf, pl.ds(q * QCH, QCH), :],
            sems.at[buf, q],
        )

    @pl.when(step == 0)
    def _():
        for j in range(NBUF):
            for q in range(QS):
                copy_obj(j, j, q).start()

    buf = jax.lax.rem(step, NBUF)
    w = w_ref[...]
    for j in range(NBUF):
        @pl.when(buf == j)
        def _(j=j):
            for q in range(QS):
                copy_obj(step, j, q).wait()
                logits = jax.lax.dot_general(
                    xbuf[j, q * QCH:(q + 1) * QCH, :], w,
                    (((1,), (1,)), ((), ())),
                    preferred_element_type=jnp.float32,
                )
                _top2(logits, g_ref, i_ref, q * QCH)

                @pl.when(step + NBUF < nsteps)
                def _():
                    copy_obj(step + NBUF, j, q).start()


@jax.jit
def _router(x, W):
    n, d = x.shape
    num_e = W.shape[0]
    grid = (n // CH,)
    gates, idx = pl.pallas_call(
        _router_body,
        grid=grid,
        in_specs=[
            pl.BlockSpec(memory_space=pl.ANY),
            pl.BlockSpec((num_e, d), lambda t: (0, 0)),
        ],
        out_specs=[
            pl.BlockSpec((CH, 2), lambda t: (t, 0)),
            pl.BlockSpec((CH, 2), lambda t: (t, 0)),
        ],
        out_shape=[
            jax.ShapeDtypeStruct((n, 2), jnp.float32),
            jax.ShapeDtypeStruct((n, 2), jnp.int32),
        ],
        scratch_shapes=[
            pltpu.VMEM((NBUF, CH, d), jnp.float32),
            pltpu.SemaphoreType.DMA((NBUF, QS)),
        ],
        compiler_params=pltpu.CompilerParams(
            dimension_semantics=("arbitrary",),
        ),
    )(x, W)
    return gates, idx


def kernel(x, W):
    gates, idx = _router(x, W)
    return gates, idx, jnp.zeros((), dtype=jnp.float32)
